# baseline (speedup 1.0000x reference)
import functools

import jax
import jax.numpy as jnp
from jax import lax
from jax.experimental import pallas as pl
from jax.experimental.pallas import tpu as pltpu
from jax.experimental.pallas import tpu_sc as plsc

_CPAD = 128


def _sc_gather_y(indexes, labels):
    b = indexes.shape[0]
    nw = 32
    bpw = b // nw
    mesh = plsc.VectorSubcoreMesh(core_axis_name="c", subcore_axis_name="s")

    @functools.partial(
        pl.kernel,
        mesh=mesh,
        out_type=jax.ShapeDtypeStruct((b,), jnp.int32),
        scratch_types=[
            pltpu.VMEM((bpw,), jnp.int32),
            pltpu.VMEM((bpw,), jnp.int32),
            pltpu.SemaphoreType.DMA,
        ],
    )
    def k(idx_hbm, lab_hbm, y_hbm, idx_v, y_v, sem):
        wid = lax.axis_index("s") * 2 + lax.axis_index("c")
        base = wid * bpw
        pltpu.sync_copy(idx_hbm.at[pl.ds(base, bpw)], idx_v)
        pltpu.async_copy(lab_hbm.at[idx_v], y_v, sem).wait()
        pltpu.sync_copy(y_v, y_hbm.at[pl.ds(base, bpw)])

    return k(indexes, labels)


def _tc_pass1(x, labels_col, idx_col, block_w):
    b, n = x.shape
    nblk = pl.cdiv(n, block_w)

    def body(x_ref, lab_ref, idx_ref, z_out, t_out, z_acc, t_acc):
        k = pl.program_id(0)

        @pl.when(k == 0)
        def _init():
            z_acc[...] = jnp.zeros_like(z_acc)
            t_acc[...] = jnp.zeros_like(t_acc)

        cols = lax.broadcasted_iota(jnp.int32, (1, block_w), 1) + k * block_w
        e = jnp.exp(x_ref[...])
        kill = (cols == idx_ref[...]) | (cols >= n)
        e0 = jnp.where(kill, 0.0, e)
        onehot = (lab_ref[...] ==
                  lax.broadcasted_iota(jnp.int32, (1, _CPAD), 1))
        z_acc[...] += jnp.sum(e0, axis=1, keepdims=True)
        t_acc[...] += jnp.dot(e0.astype(jnp.bfloat16),
                              onehot.astype(jnp.bfloat16),
                              preferred_element_type=jnp.float32)

        @pl.when(k == nblk - 1)
        def _out():
            z_out[...] = z_acc[...]
            t_out[...] = t_acc[...]

    return pl.pallas_call(
        body,
        grid=(nblk,),
        in_specs=[
            pl.BlockSpec((b, block_w), lambda k: (0, k)),
            pl.BlockSpec((block_w, 1), lambda k: (k, 0)),
            pl.BlockSpec((b, 1), lambda k: (0, 0)),
        ],
        out_specs=[
            pl.BlockSpec((b, 1), lambda k: (0, 0)),
            pl.BlockSpec((b, _CPAD), lambda k: (0, 0)),
        ],
        out_shape=[
            jax.ShapeDtypeStruct((b, 1), jnp.float32),
            jax.ShapeDtypeStruct((b, _CPAD), jnp.float32),
        ],
        scratch_shapes=[
            pltpu.VMEM((b, 1), jnp.float32),
            pltpu.VMEM((b, _CPAD), jnp.float32),
        ],
        compiler_params=pltpu.CompilerParams(
            dimension_semantics=("arbitrary",),
        ),
    )(x, labels_col, idx_col)


def _tc_finalize(z_col, t_mat, y_col):
    b = z_col.shape[0]

    def body(z_ref, t_ref, y_ref, out_ref):
        onehot = y_ref[...] == lax.broadcasted_iota(jnp.int32, (1, _CPAD), 1)
        p = jnp.sum(jnp.where(onehot, t_ref[...], 0.0), axis=1, keepdims=True)
        prob = p / z_ref[...]
        nz = prob != 0.0
        terms = jnp.where(nz, jnp.log(jnp.where(nz, prob, 1.0)), 0.0)
        out_ref[0, 0] = -jnp.sum(terms) / jnp.float32(b)

    out = pl.pallas_call(
        body,
        out_specs=pl.BlockSpec(memory_space=pltpu.SMEM),
        out_shape=jax.ShapeDtypeStruct((1, 1), jnp.float32),
    )(z_col, t_mat, y_col)
    return out[0, 0]


def kernel(x, indexes, labels):
    b, n = x.shape
    y = _sc_gather_y(indexes, labels)
    z_col, t_mat = _tc_pass1(x, labels.reshape(n, 1),
                             indexes.reshape(b, 1), block_w=2048)
    return _tc_finalize(z_col, t_mat, y.reshape(b, 1))


# speedup vs baseline: 3.0739x; 3.0739x over previous
import functools

import jax
import jax.numpy as jnp
from jax import lax
from jax.experimental import pallas as pl
from jax.experimental.pallas import tpu as pltpu
from jax.experimental.pallas import tpu_sc as plsc

_CPAD = 128


def _sc_gather_y(indexes, labels):
    b = indexes.shape[0]
    nw = 32
    bpw = b // nw
    mesh = plsc.VectorSubcoreMesh(core_axis_name="c", subcore_axis_name="s")

    @functools.partial(
        pl.kernel,
        mesh=mesh,
        out_type=jax.ShapeDtypeStruct((b,), jnp.int32),
        scratch_types=[
            pltpu.VMEM((bpw,), jnp.int32),
            pltpu.VMEM((bpw,), jnp.int32),
            pltpu.SemaphoreType.DMA,
        ],
    )
    def k(idx_hbm, lab_hbm, y_hbm, idx_v, y_v, sem):
        wid = lax.axis_index("s") * 2 + lax.axis_index("c")
        base = wid * bpw
        pltpu.sync_copy(idx_hbm.at[pl.ds(base, bpw)], idx_v)
        pltpu.async_copy(lab_hbm.at[idx_v], y_v, sem).wait()
        pltpu.sync_copy(y_v, y_hbm.at[pl.ds(base, bpw)])

    return k(indexes, labels)


def _nca_tc(xt, labels_row, y_row, idx_row, block_n):
    n, b = xt.shape
    nblk = pl.cdiv(n, block_n)

    def body(xt_ref, lab_ref, y_ref, idx_ref, out_ref, z_acc, t_acc):
        k = pl.program_id(0)
        c_iota = lax.broadcasted_iota(jnp.int32, (_CPAD, 1), 0)

        @pl.when(k == 0)
        def _init():
            z_acc[...] = jnp.zeros_like(z_acc)
            t_acc[...] = jnp.zeros_like(t_acc)

        rows = lax.broadcasted_iota(jnp.int32, (block_n, 1), 0) + k * block_n
        e = jnp.exp(xt_ref[...])
        kill = (rows == idx_ref[...]) | (rows >= n)
        e0 = jnp.where(kill, 0.0, e)
        onehot = (lab_ref[...] == c_iota).astype(jnp.bfloat16)
        z_acc[...] += jnp.sum(e0, axis=0, keepdims=True)
        t_acc[...] += jnp.dot(onehot, e0.astype(jnp.bfloat16),
                              preferred_element_type=jnp.float32)

        @pl.when(k == nblk - 1)
        def _last():
            sel = y_ref[...] == c_iota
            p = jnp.sum(jnp.where(sel, t_acc[...], 0.0), axis=0, keepdims=True)
            prob = p / z_acc[...]
            nz = prob != 0.0
            terms = jnp.where(nz, jnp.log(jnp.where(nz, prob, 1.0)), 0.0)
            out_ref[0, 0] = -jnp.sum(terms) / jnp.float32(b)

    out = pl.pallas_call(
        body,
        grid=(nblk,),
        in_specs=[
            pl.BlockSpec((block_n, b), lambda k: (k, 0)),
            pl.BlockSpec((1, block_n), lambda k: (0, k)),
            pl.BlockSpec((1, b), lambda k: (0, 0)),
            pl.BlockSpec((1, b), lambda k: (0, 0)),
        ],
        out_specs=pl.BlockSpec((1, 1), lambda k: (0, 0),
                               memory_space=pltpu.SMEM),
        out_shape=jax.ShapeDtypeStruct((1, 1), jnp.float32),
        scratch_shapes=[
            pltpu.VMEM((1, b), jnp.float32),
            pltpu.VMEM((_CPAD, b), jnp.float32),
        ],
        compiler_params=pltpu.CompilerParams(
            dimension_semantics=("arbitrary",),
        ),
    )(xt, labels_row, y_row, idx_row)
    return out[0, 0]


def kernel(x, indexes, labels):
    b, n = x.shape
    y = _sc_gather_y(indexes, labels)
    return _nca_tc(x.T, labels.reshape(1, n), y.reshape(1, b),
                   indexes.reshape(1, b), block_n=2048)
